# Initial kernel scaffold; baseline (speedup 1.0000x reference)
import jax
import jax.numpy as jnp
from jax.experimental import pallas as pl


def kernel(x, edge_index, batch, W1, a1s, a1d, b1, W2, a2s, a2d, b2, W3, a3s, a3d, b3, g1, be1, g2, be2, Wf1, bf1, Wf2, bf2):
    raise NotImplementedError("write your pallas kernel here")


# speedup vs baseline: 1.0000x; 1.0000x over previous
import jax
import jax.numpy as jnp
from jax.experimental import pallas as pl


def _gat_conv(x, edge_index, W, a_src, a_dst, b, heads, out_ch, concat):
    N = x.shape[0]
    h = (x @ W).reshape(N, heads, out_ch)
    loop = jnp.arange(N, dtype=edge_index.dtype)
    src = jnp.concatenate([edge_index[0], loop])
    dst = jnp.concatenate([edge_index[1], loop])
    a_s = jnp.sum(h * a_src[None, :, :], axis=-1)
    a_d = jnp.sum(h * a_dst[None, :, :], axis=-1)
    alpha = a_s[src] + a_d[dst]
    alpha = jax.nn.leaky_relu(alpha, negative_slope=0.2)
    amax = jax.ops.segment_max(alpha, dst, num_segments=N)
    amax = jnp.where(jnp.isfinite(amax), amax, 0.0)
    ea = jnp.exp(alpha - amax[dst])
    denom = jax.ops.segment_sum(ea, dst, num_segments=N)
    coef = ea / jnp.maximum(denom[dst], 1e-16)
    out = jax.ops.segment_sum(h[src] * coef[:, :, None], dst, num_segments=N)
    if concat:
        out = out.reshape(N, heads * out_ch)
    else:
        out = out.mean(axis=1)
    return out + b


def _bn(x, g, b):
    m = jnp.mean(x, axis=0)
    v = jnp.var(x, axis=0)
    return (x - m) / jnp.sqrt(v + 1e-5) * g + b


def _mlp_kernel(g_ref, wf1_ref, bf1_ref, wf2_ref, bf2_ref, out_ref):
    h = jnp.maximum(g_ref[...] @ wf1_ref[...] + bf1_ref[...], 0.0)
    out_ref[...] = h @ wf2_ref[...] + bf2_ref[...]


def kernel(x, edge_index, batch, W1, a1s, a1d, b1, W2, a2s, a2d, b2, W3, a3s, a3d, b3, g1, be1, g2, be2, Wf1, bf1, Wf2, bf2):
    N_GRAPHS = 64
    x1 = jax.nn.elu(_bn(_gat_conv(x, edge_index, W1, a1s, a1d, b1, 4, 32, True), g1, be1))
    x2 = jax.nn.elu(_bn(_gat_conv(x1, edge_index, W2, a2s, a2d, b2, 4, 64, True), g2, be2))
    x3 = jax.nn.elu(_gat_conv(x2, edge_index, W3, a3s, a3d, b3, 2, 128, False))
    cnt = jnp.maximum(jax.ops.segment_sum(jnp.ones((x3.shape[0],), x3.dtype), batch, num_segments=N_GRAPHS), 1.0)
    meanp = jax.ops.segment_sum(x3, batch, num_segments=N_GRAPHS) / cnt[:, None]
    maxp = jax.ops.segment_max(x3, batch, num_segments=N_GRAPHS)
    maxp = jnp.where(jnp.isfinite(maxp), maxp, 0.0)
    g = jnp.concatenate([meanp, maxp], axis=1)
    out = pl.pallas_call(
        _mlp_kernel,
        out_shape=jax.ShapeDtypeStruct((N_GRAPHS, 10), jnp.float32),
    )(g, Wf1, bf1, Wf2, bf2)
    return out


# speedup vs baseline: 3.2135x; 3.2135x over previous
import functools
import jax
import jax.numpy as jnp
from jax import lax
from jax.experimental import pallas as pl
from jax.experimental.pallas import tpu as pltpu
from jax.experimental.pallas import tpu_sc as plsc

N = 50000
NG = 64
RNG = 112
NRANGE = 448
NRI = NRANGE // 32
EB = 128
NTILE = 16
E_TOT = 800000 + N
E_PAD = 907008
BLK = 1000
NB = N // BLK
f32 = jnp.float32
i32 = jnp.int32


def _scores_epilogue(h, as_ref, ad_ref, ht_ref, adt_ref, mp_ref, i, H, HC, ROW):
    B = h.shape[0]
    a_s = jnp.dot(h, as_ref[...], preferred_element_type=f32)
    a_d = jnp.dot(h, ad_ref[...], preferred_element_type=f32)
    ht_ref[:, 0:HC] = h
    ht_ref[:, HC:HC + H] = a_s
    ht_ref[:, HC + H:ROW] = jnp.zeros((B, ROW - HC - H), f32)
    adt_ref[:, 0:H] = a_d
    adt_ref[:, H:16] = jnp.zeros((B, 16 - H), f32)
    _mp_update(a_s, a_d, mp_ref, i, H)


def _mp_update(a_s, a_d, mp_ref, i, H):

    @pl.when(i == 0)
    def _():
        mp_ref[...] = jnp.full((8, 128), -1e30, f32)

    m1 = jnp.max(a_s, axis=0)
    m2 = jnp.max(a_d, axis=0)
    mp_ref[:, 0:H] = jnp.maximum(mp_ref[:, 0:H], jnp.broadcast_to(m1[None, :], (8, H)))
    mp_ref[:, 8:8 + H] = jnp.maximum(mp_ref[:, 8:8 + H], jnp.broadcast_to(m2[None, :], (8, H)))


def _front(x, W, As, Ad, H, Cc):
    HC = H * Cc
    ROW = HC + 128

    def body(x_ref, w_ref, as_ref, ad_ref, ht_ref, adt_ref, mp_ref):
        i = pl.program_id(0)
        h = jnp.dot(x_ref[...], w_ref[...], preferred_element_type=f32)
        _scores_epilogue(h, as_ref, ad_ref, ht_ref, adt_ref, mp_ref, i, H, HC, ROW)

    return pl.pallas_call(
        body,
        grid=(NB,),
        in_specs=[
            pl.BlockSpec((BLK, x.shape[1]), lambda i: (i, 0)),
            pl.BlockSpec(W.shape, lambda i: (0, 0)),
            pl.BlockSpec(As.shape, lambda i: (0, 0)),
            pl.BlockSpec(Ad.shape, lambda i: (0, 0)),
        ],
        out_specs=[
            pl.BlockSpec((BLK, ROW), lambda i: (i, 0)),
            pl.BlockSpec((BLK, 16), lambda i: (i, 0)),
            pl.BlockSpec((8, 128), lambda i: (0, 0)),
        ],
        out_shape=[
            jax.ShapeDtypeStruct((N, ROW), f32),
            jax.ShapeDtypeStruct((N, 16), f32),
            jax.ShapeDtypeStruct((8, 128), f32),
        ],
    )(x, W, As, Ad)


def _stats(yfull, Cin):
    RIN = yfull.shape[1]

    def body(y_ref, st_ref):
        i = pl.program_id(0)
        y = y_ref[...][:, 0:Cin]

        @pl.when(i == 0)
        def _():
            st_ref[...] = jnp.zeros((8, Cin), f32)

        st_ref[0:1, :] = st_ref[0:1, :] + jnp.sum(y, axis=0)[None, :]
        st_ref[1:2, :] = st_ref[1:2, :] + jnp.sum(y * y, axis=0)[None, :]

    return pl.pallas_call(
        body,
        grid=(NB,),
        in_specs=[pl.BlockSpec((BLK, RIN), lambda i: (i, 0))],
        out_specs=[pl.BlockSpec((8, Cin), lambda i: (0, 0))],
        out_shape=[jax.ShapeDtypeStruct((8, Cin), f32)],
    )(yfull)[0]


def _normmm(yfull, scale8, shift8, W, As, Ad, Cin, H, Cc):
    HC = H * Cc
    ROW = HC + 128
    RIN = yfull.shape[1]

    def body(y_ref, sc_ref, sh_ref, w_ref, as_ref, ad_ref, ht_ref, adt_ref, mp_ref):
        i = pl.program_id(0)
        y = y_ref[...][:, 0:Cin]
        xn = y * sc_ref[0:1, :] + sh_ref[0:1, :]
        x1 = jnp.where(xn > 0, xn, jnp.exp(xn) - 1.0)
        h = jnp.dot(x1, w_ref[...], preferred_element_type=f32)
        _scores_epilogue(h, as_ref, ad_ref, ht_ref, adt_ref, mp_ref, i, H, HC, ROW)

    return pl.pallas_call(
        body,
        grid=(NB,),
        in_specs=[
            pl.BlockSpec((BLK, RIN), lambda i: (i, 0)),
            pl.BlockSpec((8, Cin), lambda i: (0, 0)),
            pl.BlockSpec((8, Cin), lambda i: (0, 0)),
            pl.BlockSpec(W.shape, lambda i: (0, 0)),
            pl.BlockSpec(As.shape, lambda i: (0, 0)),
            pl.BlockSpec(Ad.shape, lambda i: (0, 0)),
        ],
        out_specs=[
            pl.BlockSpec((BLK, ROW), lambda i: (i, 0)),
            pl.BlockSpec((BLK, 16), lambda i: (i, 0)),
            pl.BlockSpec((8, 128), lambda i: (0, 0)),
        ],
        out_shape=[
            jax.ShapeDtypeStruct((N, ROW), f32),
            jax.ShapeDtypeStruct((N, 16), f32),
            jax.ShapeDtypeStruct((8, 128), f32),
        ],
    )(yfull, scale8, shift8, W, As, Ad)


def _pool_mlp(x3, batch3, batch_col, Wf1, bf1r, Wf2, bf2r):
    def body(x_ref, b_ref, bc_ref, wf1_ref, bf1_ref, wf2_ref, bf2_ref,
             out_ref, macc_ref, cacc_ref, mx_ref):
        i = pl.program_id(0)
        x = x_ref[...]
        b = b_ref[...].reshape(BLK)
        bc = bc_ref[...]
        SfT = (lax.broadcasted_iota(i32, (NG, BLK), 0) == b[None, :]).astype(f32)

        @pl.when(i == 0)
        def _():
            macc_ref[...] = jnp.zeros((NG, 128), f32)
            cacc_ref[...] = jnp.zeros((NG, 8), f32)
            mx_ref[...] = jnp.full((NG, 128), -1e30, f32)

        macc_ref[...] = macc_ref[...] + jnp.dot(
            SfT, x, preferred_element_type=f32)
        cacc_ref[...] = cacc_ref[...] + jnp.dot(
            SfT, jnp.ones((BLK, 8), f32), preferred_element_type=f32)

        glo = jnp.min(b)
        ghi = jnp.max(b)
        for g in range(NG):
            @pl.when(jnp.logical_and(g >= glo, g <= ghi))
            def _(g=g):
                mg = jnp.max(jnp.where(bc == g, x, -1e30), axis=0)
                mx_ref[g:g + 1, :] = jnp.maximum(mx_ref[g:g + 1, :], mg[None, :])

        @pl.when(i == NB - 1)
        def _():
            cnt = cacc_ref[:, 0:1]
            meanp = macc_ref[...] / jnp.maximum(cnt, 1.0)
            maxp = jnp.where(cnt > 0, mx_ref[...], 0.0)
            gcat = jnp.concatenate([meanp, maxp], axis=1)
            hm = jnp.maximum(
                jnp.dot(gcat, wf1_ref[...], preferred_element_type=f32)
                + bf1_ref[0:1, :], 0.0)
            out_ref[...] = (jnp.dot(hm, wf2_ref[...], preferred_element_type=f32)
                            + bf2_ref[0:1, :])

    outs = pl.pallas_call(
        body,
        grid=(NB,),
        in_specs=[
            pl.BlockSpec((BLK, 128), lambda i: (i, 0)),
            pl.BlockSpec((1, 1, BLK), lambda i: (i, 0, 0)),
            pl.BlockSpec((BLK, 1), lambda i: (i, 0)),
            pl.BlockSpec(Wf1.shape, lambda i: (0, 0)),
            pl.BlockSpec((8, 128), lambda i: (0, 0)),
            pl.BlockSpec(Wf2.shape, lambda i: (0, 0)),
            pl.BlockSpec((8, 16), lambda i: (0, 0)),
        ],
        out_specs=[
            pl.BlockSpec((NG, 16), lambda i: (0, 0)),
            pl.BlockSpec((NG, 128), lambda i: (0, 0)),
            pl.BlockSpec((NG, 8), lambda i: (0, 0)),
            pl.BlockSpec((NG, 128), lambda i: (0, 0)),
        ],
        out_shape=[
            jax.ShapeDtypeStruct((NG, 16), f32),
            jax.ShapeDtypeStruct((NG, 128), f32),
            jax.ShapeDtypeStruct((NG, 8), f32),
            jax.ShapeDtypeStruct((NG, 128), f32),
        ],
    )(x3, batch3, batch_col, Wf1, bf1r, Wf2, bf2r)
    return outs[0]


def _sget(vref, idx):
    v = plsc.load_gather(vref, [jnp.full((16,), idx, i32)])
    return jnp.max(v)


def _splat(vref, r, c):
    return plsc.load_gather(vref, [jnp.full((16,), r, i32), jnp.full((16,), c, i32)])


def _make_sc_layer(H, Cc, last):
    HC = H * Cc
    ROW = HC + 128
    OUTC = Cc if last else ROW
    mesh = plsc.VectorSubcoreMesh(core_axis_name="c", subcore_axis_name="s",
                                  num_cores=2, num_subcores=NTILE)

    @functools.partial(
        pl.kernel,
        out_type=jax.ShapeDtypeStruct((N, OUTC), f32),
        mesh=mesh,
        compiler_params=pltpu.CompilerParams(needs_layout_passes=False),
        scratch_types=[
            pltpu.VMEM((EB,), i32),
            pltpu.VMEM((EB,), i32),
            pltpu.VMEM((2 * NRANGE,), i32),
            pltpu.VMEM((EB, ROW), f32),
            pltpu.VMEM(((RNG + 9) * 16,), f32),
            pltpu.VMEM((EB * 16,), f32),
            pltpu.VMEM((RNG + 8, ROW), f32),
            pltpu.VMEM((16, 128), f32),
            pltpu.VMEM((16,), f32),
            pltpu.VMEM((128,), f32),
            pltpu.SemaphoreType.DMA,
        ],
    )
    def k(htab, adtab, srcp, dlp, cinfo, mvec, bvec, out,
          srcv, dlv, cinfo_v, hbuf, adtile, eabuf, tacc, o3buf,
          mv_v, bv_v, sem):
        kc = lax.axis_index("c")
        s = lax.axis_index("s")
        wid = s * 2 + kc

        pltpu.sync_copy(cinfo, cinfo_v)
        pltpu.sync_copy(mvec, mv_v)
        pltpu.sync_copy(bvec, bv_v)
        mv = mv_v[...]
        z16 = jnp.zeros((16,), f32)

        def zea(e, _):
            eabuf[pl.ds(pl.multiple_of(e * 16, 16), 16)] = z16
            return 0
        lax.fori_loop(0, EB, zea, 0)

        def range_body(i, _):
            r = i * 32 + wid
            rs = _sget(cinfo_v, r)
            nbl = _sget(cinfo_v, NRANGE + r)
            nrows = jnp.minimum(RNG, N - r * RNG)

            def zrow(rr, _):
                def zcol(j, _):
                    tacc[rr, pl.ds(pl.multiple_of(j * 16, 16), 16)] = z16
                    return 0
                lax.fori_loop(0, ROW // 16, zcol, 0)
                return 0
            lax.fori_loop(0, RNG + 8, zrow, 0)

            arow = pl.multiple_of(r * RNG * 16, 128)
            pltpu.sync_copy(adtab.at[pl.ds(arow, RNG * 16)],
                            adtile.at[pl.ds(0, RNG * 16)])
            adtile[pl.ds(RNG * 16, 16)] = jnp.full((16,), -1e30, f32)

            def blk(j, _):
                eo = pl.multiple_of(rs + j * EB, EB)
                pltpu.sync_copy(srcp.at[pl.ds(eo, EB)], srcv)
                pltpu.sync_copy(dlp.at[pl.ds(eo, EB)], dlv)
                pltpu.async_copy(htab.at[srcv], hbuf, sem).wait()

                def grp(g, _):
                    e16 = g * 16 + lax.iota(i32, 16)
                    dl16 = dlv[pl.ds(pl.multiple_of(g * 16, 16), 16)]
                    for h in range(H):
                        asv = plsc.load_gather(
                            hbuf, [e16, jnp.full((16,), HC + h, i32)])
                        adv = plsc.load_gather(adtile, [dl16 * 16 + h])
                        al = asv + adv
                        al = jnp.where(al > 0, al, 0.2 * al)
                        ea = jnp.exp(al - mv)
                        plsc.store_scatter(eabuf, [e16 * 16 + h], ea)
                    return 0
                lax.fori_loop(0, EB // 16, grp, 0)

                def edge(e, _):
                    dloc = _sget(dlv, e)
                    ee = pl.multiple_of(e * 16, 16)
                    tacc[dloc, pl.ds(HC, 16)] = (
                        tacc[dloc, pl.ds(HC, 16)] + eabuf[pl.ds(ee, 16)])
                    for h in range(H):
                        easp = plsc.load_gather(
                            eabuf, [jnp.full((16,), e * 16 + h, i32)])
                        for jc in range(Cc // 16):
                            off = h * Cc + jc * 16
                            tacc[dloc, pl.ds(off, 16)] = (
                                tacc[dloc, pl.ds(off, 16)]
                                + hbuf[e, pl.ds(off, 16)] * easp)
                    return 0
                lax.fori_loop(0, EB, edge, 0)
                return 0
            lax.fori_loop(0, nbl, blk, 0)

            def ep(t16, _):
                r0b = pl.multiple_of(t16 * 16, 16)

                @pl.when(r0b < nrows)
                def _():
                    def row(rr2, _):
                        rr = r0b + rr2
                        if not last:
                            for h in range(H):
                                rec = 1.0 / jnp.maximum(
                                    _splat(tacc, rr, HC + h), 1e-30)
                                for jc in range(Cc // 16):
                                    off = h * Cc + jc * 16
                                    tacc[rr, pl.ds(off, 16)] = (
                                        tacc[rr, pl.ds(off, 16)] * rec)
                        else:
                            r0h = 1.0 / jnp.maximum(
                                _splat(tacc, rr, HC + 0), 1e-30)
                            r1h = 1.0 / jnp.maximum(
                                _splat(tacc, rr, HC + 1), 1e-30)
                            for jc in range(Cc // 16):
                                off = jc * 16
                                v0 = tacc[rr, pl.ds(off, 16)] * r0h
                                v1 = tacc[rr, pl.ds(Cc + off, 16)] * r1h
                                m = (v0 + v1) * 0.5 + bv_v[pl.ds(off, 16)]
                                m = jnp.where(m > 0, m, jnp.exp(m) - 1.0)
                                o3buf[rr2, pl.ds(off, 16)] = m
                        return 0
                    lax.fori_loop(0, 16, row, 0)
                    drow = pl.multiple_of(r * RNG + r0b, 16)
                    if last:
                        pltpu.sync_copy(o3buf, out.at[pl.ds(drow, 16)])
                    else:
                        pltpu.sync_copy(tacc.at[pl.ds(r0b, 16)],
                                        out.at[pl.ds(drow, 16)])
                return 0
            lax.fori_loop(0, RNG // 16, ep, 0)
            return 0
        lax.fori_loop(0, NRI, range_body, 0)

    return k


def _expand_att(a):
    H, Cc = a.shape
    return (jnp.eye(H, dtype=f32)[:, None, :] * a[:, :, None]).reshape(H * Cc, H)


def _mvec_from_mp(mp, H):
    mas = mp[0, 0:H]
    mad = mp[0, 8:8 + H]
    M = jnp.maximum(jnp.max(mas + mad), 0.0)
    return jnp.full((16,), M, f32)


def _scale_shift(st, Cin, g, be):
    s1 = st[0]
    s2 = st[1]
    mean = s1 / N
    var = s2 / N - mean * mean
    rstd = 1.0 / jnp.sqrt(var + 1e-5)
    scale = g * rstd
    shift = be - mean * scale
    return (jnp.broadcast_to(scale[None, :], (8, Cin)),
            jnp.broadcast_to(shift[None, :], (8, Cin)))


def kernel(x, edge_index, batch, W1, a1s, a1d, b1, W2, a2s, a2d, b2,
           W3, a3s, a3d, b3, g1, be1, g2, be2, Wf1, bf1, Wf2, bf2):
    loop = jnp.arange(N, dtype=i32)
    src = jnp.concatenate([edge_index[0].astype(i32), loop])
    dst = jnp.concatenate([edge_index[1].astype(i32), loop])
    perm = jnp.argsort(dst)
    src_s = src[perm]
    dst_s = dst[perm]
    ci = dst_s // RNG
    counts = jnp.bincount(ci, length=NRANGE).astype(i32)
    starts = jnp.concatenate(
        [jnp.zeros((1,), i32), jnp.cumsum(counts)[:-1].astype(i32)])
    plen = ((counts + EB - 1) // EB) * EB
    pstart = jnp.concatenate(
        [jnp.zeros((1,), i32), jnp.cumsum(plen)[:-1].astype(i32)])
    js = jnp.arange(E_PAD, dtype=i32)
    cs = (jnp.searchsorted(pstart, js, side='right') - 1).astype(i32)
    off = js - pstart[cs]
    srci = jnp.minimum(starts[cs] + off, E_TOT - 1)
    valid = off < counts[cs]
    sv = src_s[srci]
    dv = dst_s[srci]
    src_p = jnp.where(valid, sv, 0).astype(i32)
    dlp = jnp.where(valid, dv - cs * RNG, RNG).astype(i32)
    cnbt = (plen // EB).astype(i32)
    cinfo = jnp.concatenate([pstart, cnbt])
    zb = jnp.zeros((128,), f32)

    adpad = jnp.zeros((NRANGE * RNG - N, 16), f32)

    def _flat_ad(adt):
        return jnp.concatenate([adt, adpad], axis=0).reshape(NRANGE * RNG * 16)

    ht1, adt1, mp1 = _front(x, W1, _expand_att(a1s), _expand_att(a1d), 4, 32)
    adt1 = _flat_ad(adt1)
    y1 = _make_sc_layer(4, 32, False)(
        ht1, adt1, src_p, dlp, cinfo, _mvec_from_mp(mp1, 4), zb)

    st1 = _stats(y1, 128)
    sc1, sh1 = _scale_shift(st1, 128, g1, be1)
    ht2, adt2, mp2 = _normmm(y1, sc1, sh1, W2, _expand_att(a2s),
                             _expand_att(a2d), 128, 4, 64)
    adt2 = _flat_ad(adt2)
    y2 = _make_sc_layer(4, 64, False)(
        ht2, adt2, src_p, dlp, cinfo, _mvec_from_mp(mp2, 4), zb)

    st2 = _stats(y2, 256)
    sc2, sh2 = _scale_shift(st2, 256, g2, be2)
    ht3, adt3, mp3 = _normmm(y2, sc2, sh2, W3, _expand_att(a3s),
                             _expand_att(a3d), 256, 2, 128)
    adt3 = _flat_ad(adt3)
    x3 = _make_sc_layer(2, 128, True)(
        ht3, adt3, src_p, dlp, cinfo, _mvec_from_mp(mp3, 2), b3)

    batch3 = batch.astype(i32).reshape(NB, 1, BLK)
    batch_col = batch.astype(i32).reshape(N, 1)
    bf1r = jnp.broadcast_to(bf1[None, :], (8, 128))
    bf2r = jnp.broadcast_to(
        jnp.concatenate([bf2, jnp.zeros((6,), f32)])[None, :], (8, 16))
    Wf2p = jnp.concatenate([Wf2, jnp.zeros((128, 6), f32)], axis=1)
    out16 = _pool_mlp(x3, batch3, batch_col, Wf1, bf1r, Wf2p, bf2r)
    return out16[:, 0:10]


# speedup vs baseline: 21.0898x; 6.5628x over previous
import functools
import jax
import jax.numpy as jnp
from jax import lax
from jax.experimental import pallas as pl
from jax.experimental.pallas import tpu as pltpu
from jax.experimental.pallas import tpu_sc as plsc

N = 50000
NG = 64
RNG = 112
NRANGE = 448
NRI = NRANGE // 32
EB = 128
NTILE = 16
E_TOT = 800000 + N
E_PAD = 907008
BLK = 1000
NB = N // BLK
f32 = jnp.float32
i32 = jnp.int32


def _scores_epilogue(h, as_ref, ad_ref, ht_ref, adt_ref, mp_ref, i, H, HC, ROW):
    B = h.shape[0]
    a_s = jnp.dot(h, as_ref[...], preferred_element_type=f32)
    a_d = jnp.dot(h, ad_ref[...], preferred_element_type=f32)
    ht_ref[:, 0:HC] = h
    ht_ref[:, HC:HC + H] = a_s
    ht_ref[:, HC + H:ROW] = jnp.zeros((B, ROW - HC - H), f32)
    adt_ref[:, 0:H] = a_d
    adt_ref[:, H:16] = jnp.zeros((B, 16 - H), f32)
    _mp_update(a_s, a_d, mp_ref, i, H)


def _mp_update(a_s, a_d, mp_ref, i, H):

    @pl.when(i == 0)
    def _():
        mp_ref[...] = jnp.full((8, 128), -1e30, f32)

    m1 = jnp.max(a_s, axis=0)
    m2 = jnp.max(a_d, axis=0)
    mp_ref[:, 0:H] = jnp.maximum(mp_ref[:, 0:H], jnp.broadcast_to(m1[None, :], (8, H)))
    mp_ref[:, 8:8 + H] = jnp.maximum(mp_ref[:, 8:8 + H], jnp.broadcast_to(m2[None, :], (8, H)))


def _front(x, W, As, Ad, H, Cc):
    HC = H * Cc
    ROW = HC + 128

    def body(x_ref, w_ref, as_ref, ad_ref, ht_ref, adt_ref, mp_ref):
        i = pl.program_id(0)
        h = jnp.dot(x_ref[...], w_ref[...], preferred_element_type=f32)
        _scores_epilogue(h, as_ref, ad_ref, ht_ref, adt_ref, mp_ref, i, H, HC, ROW)

    return pl.pallas_call(
        body,
        grid=(NB,),
        in_specs=[
            pl.BlockSpec((BLK, x.shape[1]), lambda i: (i, 0)),
            pl.BlockSpec(W.shape, lambda i: (0, 0)),
            pl.BlockSpec(As.shape, lambda i: (0, 0)),
            pl.BlockSpec(Ad.shape, lambda i: (0, 0)),
        ],
        out_specs=[
            pl.BlockSpec((BLK, ROW), lambda i: (i, 0)),
            pl.BlockSpec((BLK, 16), lambda i: (i, 0)),
            pl.BlockSpec((8, 128), lambda i: (0, 0)),
        ],
        out_shape=[
            jax.ShapeDtypeStruct((N, ROW), f32),
            jax.ShapeDtypeStruct((N, 16), f32),
            jax.ShapeDtypeStruct((8, 128), f32),
        ],
    )(x, W, As, Ad)


def _stats(yfull, Cin):
    RIN = yfull.shape[1]

    def body(y_ref, st_ref):
        i = pl.program_id(0)
        y = y_ref[...][:, 0:Cin]

        @pl.when(i == 0)
        def _():
            st_ref[...] = jnp.zeros((8, Cin), f32)

        st_ref[0:1, :] = st_ref[0:1, :] + jnp.sum(y, axis=0)[None, :]
        st_ref[1:2, :] = st_ref[1:2, :] + jnp.sum(y * y, axis=0)[None, :]

    return pl.pallas_call(
        body,
        grid=(NB,),
        in_specs=[pl.BlockSpec((BLK, RIN), lambda i: (i, 0))],
        out_specs=[pl.BlockSpec((8, Cin), lambda i: (0, 0))],
        out_shape=[jax.ShapeDtypeStruct((8, Cin), f32)],
    )(yfull)[0]


def _normmm(yfull, scale8, shift8, W, As, Ad, Cin, H, Cc):
    HC = H * Cc
    ROW = HC + 128
    RIN = yfull.shape[1]

    def body(y_ref, sc_ref, sh_ref, w_ref, as_ref, ad_ref, ht_ref, adt_ref, mp_ref):
        i = pl.program_id(0)
        y = y_ref[...][:, 0:Cin]
        xn = y * sc_ref[0:1, :] + sh_ref[0:1, :]
        x1 = jnp.where(xn > 0, xn, jnp.exp(xn) - 1.0)
        h = jnp.dot(x1, w_ref[...], preferred_element_type=f32)
        _scores_epilogue(h, as_ref, ad_ref, ht_ref, adt_ref, mp_ref, i, H, HC, ROW)

    return pl.pallas_call(
        body,
        grid=(NB,),
        in_specs=[
            pl.BlockSpec((BLK, RIN), lambda i: (i, 0)),
            pl.BlockSpec((8, Cin), lambda i: (0, 0)),
            pl.BlockSpec((8, Cin), lambda i: (0, 0)),
            pl.BlockSpec(W.shape, lambda i: (0, 0)),
            pl.BlockSpec(As.shape, lambda i: (0, 0)),
            pl.BlockSpec(Ad.shape, lambda i: (0, 0)),
        ],
        out_specs=[
            pl.BlockSpec((BLK, ROW), lambda i: (i, 0)),
            pl.BlockSpec((BLK, 16), lambda i: (i, 0)),
            pl.BlockSpec((8, 128), lambda i: (0, 0)),
        ],
        out_shape=[
            jax.ShapeDtypeStruct((N, ROW), f32),
            jax.ShapeDtypeStruct((N, 16), f32),
            jax.ShapeDtypeStruct((8, 128), f32),
        ],
    )(yfull, scale8, shift8, W, As, Ad)


def _pool_mlp(x3, batch3, batch_col, Wf1, bf1r, Wf2, bf2r):
    def body(x_ref, b_ref, bc_ref, wf1_ref, bf1_ref, wf2_ref, bf2_ref,
             out_ref, macc_ref, cacc_ref, mx_ref):
        i = pl.program_id(0)
        x = x_ref[...]
        b = b_ref[...].reshape(BLK)
        bc = bc_ref[...]
        SfT = (lax.broadcasted_iota(i32, (NG, BLK), 0) == b[None, :]).astype(f32)

        @pl.when(i == 0)
        def _():
            macc_ref[...] = jnp.zeros((NG, 128), f32)
            cacc_ref[...] = jnp.zeros((NG, 8), f32)
            mx_ref[...] = jnp.full((NG, 128), -1e30, f32)

        macc_ref[...] = macc_ref[...] + jnp.dot(
            SfT, x, preferred_element_type=f32)
        cacc_ref[...] = cacc_ref[...] + jnp.dot(
            SfT, jnp.ones((BLK, 8), f32), preferred_element_type=f32)

        glo = jnp.min(b)
        ghi = jnp.max(b)
        for g in range(NG):
            @pl.when(jnp.logical_and(g >= glo, g <= ghi))
            def _(g=g):
                mg = jnp.max(jnp.where(bc == g, x, -1e30), axis=0)
                mx_ref[g:g + 1, :] = jnp.maximum(mx_ref[g:g + 1, :], mg[None, :])

        @pl.when(i == NB - 1)
        def _():
            cnt = cacc_ref[:, 0:1]
            meanp = macc_ref[...] / jnp.maximum(cnt, 1.0)
            maxp = jnp.where(cnt > 0, mx_ref[...], 0.0)
            gcat = jnp.concatenate([meanp, maxp], axis=1)
            hm = jnp.maximum(
                jnp.dot(gcat, wf1_ref[...], preferred_element_type=f32)
                + bf1_ref[0:1, :], 0.0)
            out_ref[...] = (jnp.dot(hm, wf2_ref[...], preferred_element_type=f32)
                            + bf2_ref[0:1, :])

    outs = pl.pallas_call(
        body,
        grid=(NB,),
        in_specs=[
            pl.BlockSpec((BLK, 128), lambda i: (i, 0)),
            pl.BlockSpec((1, 1, BLK), lambda i: (i, 0, 0)),
            pl.BlockSpec((BLK, 1), lambda i: (i, 0)),
            pl.BlockSpec(Wf1.shape, lambda i: (0, 0)),
            pl.BlockSpec((8, 128), lambda i: (0, 0)),
            pl.BlockSpec(Wf2.shape, lambda i: (0, 0)),
            pl.BlockSpec((8, 16), lambda i: (0, 0)),
        ],
        out_specs=[
            pl.BlockSpec((NG, 16), lambda i: (0, 0)),
            pl.BlockSpec((NG, 128), lambda i: (0, 0)),
            pl.BlockSpec((NG, 8), lambda i: (0, 0)),
            pl.BlockSpec((NG, 128), lambda i: (0, 0)),
        ],
        out_shape=[
            jax.ShapeDtypeStruct((NG, 16), f32),
            jax.ShapeDtypeStruct((NG, 128), f32),
            jax.ShapeDtypeStruct((NG, 8), f32),
            jax.ShapeDtypeStruct((NG, 128), f32),
        ],
    )(x3, batch3, batch_col, Wf1, bf1r, Wf2, bf2r)
    return outs[0]


def _sget(vref, idx):
    v = plsc.load_gather(vref, [jnp.full((16,), idx, i32)])
    return jnp.max(v)


def _splat(vref, r, c):
    return plsc.load_gather(vref, [jnp.full((16,), r, i32), jnp.full((16,), c, i32)])


def _make_sc_layer(H, Cc, last):
    HC = H * Cc
    ROW = HC + 128
    OUTC = Cc if last else ROW
    mesh = plsc.VectorSubcoreMesh(core_axis_name="c", subcore_axis_name="s",
                                  num_cores=2, num_subcores=NTILE)

    @functools.partial(
        pl.kernel,
        out_type=jax.ShapeDtypeStruct((N, OUTC), f32),
        mesh=mesh,
        compiler_params=pltpu.CompilerParams(needs_layout_passes=False),
        scratch_types=[
            pltpu.VMEM((EB,), i32),
            pltpu.VMEM((EB,), i32),
            pltpu.VMEM((2 * NRANGE,), i32),
            pltpu.VMEM((EB, ROW), f32),
            pltpu.VMEM(((RNG + 9) * 16,), f32),
            pltpu.VMEM((EB * 16,), f32),
            pltpu.VMEM((RNG + 8, ROW), f32),
            pltpu.VMEM((16, 128), f32),
            pltpu.VMEM((16,), f32),
            pltpu.VMEM((128,), f32),
            pltpu.SemaphoreType.DMA,
        ],
    )
    def k(htab, adtab, srcp, dlp, cinfo, mvec, bvec, out,
          srcv, dlv, cinfo_v, hbuf, adtile, eabuf, tacc, o3buf,
          mv_v, bv_v, sem):
        kc = lax.axis_index("c")
        s = lax.axis_index("s")
        wid = s * 2 + kc

        pltpu.sync_copy(cinfo, cinfo_v)
        pltpu.sync_copy(mvec, mv_v)
        pltpu.sync_copy(bvec, bv_v)
        mv = mv_v[...]
        z16 = jnp.zeros((16,), f32)

        def zea(e, _):
            eabuf[pl.ds(pl.multiple_of(e * 16, 16), 16)] = z16
            return 0
        lax.fori_loop(0, EB, zea, 0)

        def range_body(i, _):
            r = i * 32 + wid
            rs = _sget(cinfo_v, r)
            nbl = _sget(cinfo_v, NRANGE + r)
            nrows = jnp.minimum(RNG, N - r * RNG)

            def zrow(rr, _):
                def zcol(j, _):
                    tacc[rr, pl.ds(pl.multiple_of(j * 16, 16), 16)] = z16
                    return 0
                lax.fori_loop(0, ROW // 16, zcol, 0)
                return 0
            lax.fori_loop(0, RNG + 8, zrow, 0)

            arow = pl.multiple_of(r * RNG * 16, 128)
            pltpu.sync_copy(adtab.at[pl.ds(arow, RNG * 16)],
                            adtile.at[pl.ds(0, RNG * 16)])
            adtile[pl.ds(RNG * 16, 16)] = jnp.full((16,), -1e30, f32)

            def blk(j, _):
                eo = pl.multiple_of(rs + j * EB, EB)
                pltpu.sync_copy(srcp.at[pl.ds(eo, EB)], srcv)
                pltpu.sync_copy(dlp.at[pl.ds(eo, EB)], dlv)
                pltpu.async_copy(htab.at[srcv], hbuf, sem).wait()

                def grp(g, _):
                    e16 = g * 16 + lax.iota(i32, 16)
                    dl16 = dlv[pl.ds(pl.multiple_of(g * 16, 16), 16)]
                    for h in range(H):
                        asv = plsc.load_gather(
                            hbuf, [e16, jnp.full((16,), HC + h, i32)])
                        adv = plsc.load_gather(adtile, [dl16 * 16 + h])
                        al = asv + adv
                        al = jnp.where(al > 0, al, 0.2 * al)
                        ea = jnp.exp(al - mv)
                        plsc.store_scatter(eabuf, [e16 * 16 + h], ea)
                    return 0
                lax.fori_loop(0, EB // 16, grp, 0)

                def edge(e, _):
                    dloc = _sget(dlv, e)
                    ee = pl.multiple_of(e * 16, 16)
                    tacc[dloc, pl.ds(HC, 16)] = (
                        tacc[dloc, pl.ds(HC, 16)] + eabuf[pl.ds(ee, 16)])
                    for h in range(H):
                        easp = plsc.load_gather(
                            eabuf, [jnp.full((16,), e * 16 + h, i32)])
                        for jc in range(Cc // 16):
                            off = h * Cc + jc * 16
                            tacc[dloc, pl.ds(off, 16)] = (
                                tacc[dloc, pl.ds(off, 16)]
                                + hbuf[e, pl.ds(off, 16)] * easp)
                    return 0
                lax.fori_loop(0, EB, edge, 0)
                return 0
            lax.fori_loop(0, nbl, blk, 0)

            def ep(t16, _):
                r0b = pl.multiple_of(t16 * 16, 16)

                @pl.when(r0b < nrows)
                def _():
                    def row(rr2, _):
                        rr = r0b + rr2
                        if not last:
                            for h in range(H):
                                rec = 1.0 / jnp.maximum(
                                    _splat(tacc, rr, HC + h), 1e-30)
                                for jc in range(Cc // 16):
                                    off = h * Cc + jc * 16
                                    tacc[rr, pl.ds(off, 16)] = (
                                        tacc[rr, pl.ds(off, 16)] * rec)
                        else:
                            r0h = 1.0 / jnp.maximum(
                                _splat(tacc, rr, HC + 0), 1e-30)
                            r1h = 1.0 / jnp.maximum(
                                _splat(tacc, rr, HC + 1), 1e-30)
                            for jc in range(Cc // 16):
                                off = jc * 16
                                v0 = tacc[rr, pl.ds(off, 16)] * r0h
                                v1 = tacc[rr, pl.ds(Cc + off, 16)] * r1h
                                m = (v0 + v1) * 0.5 + bv_v[pl.ds(off, 16)]
                                m = jnp.where(m > 0, m, jnp.exp(m) - 1.0)
                                o3buf[rr2, pl.ds(off, 16)] = m
                        return 0
                    lax.fori_loop(0, 16, row, 0)
                    drow = pl.multiple_of(r * RNG + r0b, 16)
                    if last:
                        pltpu.sync_copy(o3buf, out.at[pl.ds(drow, 16)])
                    else:
                        pltpu.sync_copy(tacc.at[pl.ds(r0b, 16)],
                                        out.at[pl.ds(drow, 16)])
                return 0
            lax.fori_loop(0, RNG // 16, ep, 0)
            return 0
        lax.fori_loop(0, NRI, range_body, 0)

    return k


def _expand_att(a):
    H, Cc = a.shape
    return (jnp.eye(H, dtype=f32)[:, None, :] * a[:, :, None]).reshape(H * Cc, H)


def _mvec_from_mp(mp, H):
    mas = mp[0, 0:H]
    mad = mp[0, 8:8 + H]
    M = jnp.maximum(jnp.max(mas + mad), 0.0)
    return jnp.full((16,), M, f32)


def _scale_shift(st, Cin, g, be):
    s1 = st[0]
    s2 = st[1]
    mean = s1 / N
    var = s2 / N - mean * mean
    rstd = 1.0 / jnp.sqrt(var + 1e-5)
    scale = g * rstd
    shift = be - mean * scale
    return (jnp.broadcast_to(scale[None, :], (8, Cin)),
            jnp.broadcast_to(shift[None, :], (8, Cin)))


def kernel(x, edge_index, batch, W1, a1s, a1d, b1, W2, a2s, a2d, b2,
           W3, a3s, a3d, b3, g1, be1, g2, be2, Wf1, bf1, Wf2, bf2):
    loop = jnp.arange(N, dtype=i32)
    src = jnp.concatenate([edge_index[0].astype(i32), loop])
    dst = jnp.concatenate([edge_index[1].astype(i32), loop])
    perm = jnp.argsort(dst)
    src_s = src[perm]
    dst_s = dst[perm]
    ci = dst_s // RNG
    counts = jnp.bincount(ci, length=NRANGE).astype(i32)
    starts = jnp.concatenate(
        [jnp.zeros((1,), i32), jnp.cumsum(counts)[:-1].astype(i32)])
    plen = ((counts + EB - 1) // EB) * EB
    pstart = jnp.concatenate(
        [jnp.zeros((1,), i32), jnp.cumsum(plen)[:-1].astype(i32)])
    js = jnp.arange(E_PAD, dtype=i32)
    mark = jnp.zeros((E_PAD,), i32).at[pstart].add(1)
    cs = jnp.cumsum(mark).astype(i32) - 1
    padb = pstart - starts
    pend = pstart + counts
    d1 = padb - jnp.concatenate([jnp.zeros((1,), i32), padb[:-1]])
    d2 = pend - jnp.concatenate([jnp.zeros((1,), i32), pend[:-1]])
    P = jnp.cumsum(jnp.zeros((E_PAD,), i32).at[pstart].add(d1)).astype(i32)
    PE = jnp.cumsum(jnp.zeros((E_PAD,), i32).at[pstart].add(d2)).astype(i32)
    srci = jnp.minimum(js - P, E_TOT - 1)
    valid = js < PE
    sv = src_s[srci]
    dv = dst_s[srci]
    src_p = jnp.where(valid, sv, 0).astype(i32)
    dlp = jnp.where(valid, dv - cs * RNG, RNG).astype(i32)
    cnbt = (plen // EB).astype(i32)
    cinfo = jnp.concatenate([pstart, cnbt])
    zb = jnp.zeros((128,), f32)

    adpad = jnp.zeros((NRANGE * RNG - N, 16), f32)

    def _flat_ad(adt):
        return jnp.concatenate([adt, adpad], axis=0).reshape(NRANGE * RNG * 16)

    ht1, adt1, mp1 = _front(x, W1, _expand_att(a1s), _expand_att(a1d), 4, 32)
    adt1 = _flat_ad(adt1)
    y1 = _make_sc_layer(4, 32, False)(
        ht1, adt1, src_p, dlp, cinfo, _mvec_from_mp(mp1, 4), zb)

    st1 = _stats(y1, 128)
    sc1, sh1 = _scale_shift(st1, 128, g1, be1)
    ht2, adt2, mp2 = _normmm(y1, sc1, sh1, W2, _expand_att(a2s),
                             _expand_att(a2d), 128, 4, 64)
    adt2 = _flat_ad(adt2)
    y2 = _make_sc_layer(4, 64, False)(
        ht2, adt2, src_p, dlp, cinfo, _mvec_from_mp(mp2, 4), zb)

    st2 = _stats(y2, 256)
    sc2, sh2 = _scale_shift(st2, 256, g2, be2)
    ht3, adt3, mp3 = _normmm(y2, sc2, sh2, W3, _expand_att(a3s),
                             _expand_att(a3d), 256, 2, 128)
    adt3 = _flat_ad(adt3)
    x3 = _make_sc_layer(2, 128, True)(
        ht3, adt3, src_p, dlp, cinfo, _mvec_from_mp(mp3, 2), b3)

    batch3 = batch.astype(i32).reshape(NB, 1, BLK)
    batch_col = batch.astype(i32).reshape(N, 1)
    bf1r = jnp.broadcast_to(bf1[None, :], (8, 128))
    bf2r = jnp.broadcast_to(
        jnp.concatenate([bf2, jnp.zeros((6,), f32)])[None, :], (8, 16))
    Wf2p = jnp.concatenate([Wf2, jnp.zeros((128, 6), f32)], axis=1)
    out16 = _pool_mlp(x3, batch3, batch_col, Wf1, bf1r, Wf2p, bf2r)
    return out16[:, 0:10]
